# Initial kernel scaffold; baseline (speedup 1.0000x reference)
import jax
import jax.numpy as jnp
from jax.experimental import pallas as pl


def kernel(tokens, level_offsets, parent_pos, batch_index, bs, table, W, b):
    raise NotImplementedError("write your pallas kernel here")


# speedup vs baseline: 2.4314x; 2.4314x over previous
import functools

import jax
import jax.numpy as jnp
from jax import lax
from jax.experimental import pallas as pl
from jax.experimental.pallas import tpu as pltpu
from jax.experimental.pallas import tpu_sc as plsc

_LEVEL_SIZES = (16, 48, 192, 768, 3072, 8192, 10240, 10240)
_N = sum(_LEVEL_SIZES)
_D = 128
_BS = 16
_NC = 2
_NS = 16
_NW = _NC * _NS

_OFFS = [0]
for _s in _LEVEL_SIZES:
    _OFFS.append(_OFFS[-1] + _s)

_GCH = 128
_ROWS_PER_TILE = _N // _NW
_GN = _ROWS_PER_TILE // _GCH
_PCH = 256
_MMB = 512


def _mesh():
    return plsc.VectorSubcoreMesh(core_axis_name="c", subcore_axis_name="s")


@functools.partial(
    pl.kernel,
    out_type=jax.ShapeDtypeStruct((_N, _D), jnp.float32),
    mesh=_mesh(),
    scratch_types=[
        pltpu.VMEM((_GN, _GCH), jnp.int32),
        pltpu.VMEM((2, _GCH, _D), jnp.float32),
        pltpu.SemaphoreType.DMA,
        pltpu.SemaphoreType.DMA,
    ],
)
def _gather(table_hbm, tok_hbm, out_hbm, idx_v, bufs, sem0, sem1):
    wid = lax.axis_index("s") * _NC + lax.axis_index("c")
    pltpu.sync_copy(tok_hbm.at[pl.ds(wid * _GN, _GN)], idx_v)
    sems = (sem0, sem1)
    cps = [None, None]
    cps[0] = pltpu.async_copy(table_hbm.at[idx_v.at[0]], bufs.at[0], sem0)
    for c in range(_GN):
        cur = c % 2
        if c + 1 < _GN:
            cps[1 - cur] = pltpu.async_copy(
                table_hbm.at[idx_v.at[c + 1]], bufs.at[1 - cur], sems[1 - cur])
        cps[cur].wait()
        pltpu.sync_copy(
            bufs.at[cur],
            out_hbm.at[pl.ds(wid * _ROWS_PER_TILE + c * _GCH, _GCH)])


def _matmul(emb, w, b2):
    def body(x_ref, w_ref, b_ref, o_ref):
        o_ref[...] = lax.dot_general(
            x_ref[...], w_ref[...], (((1,), (1,)), ((), ())),
            preferred_element_type=jnp.float32) + b_ref[...]

    return pl.pallas_call(
        body,
        grid=(_N // _MMB,),
        in_specs=[
            pl.BlockSpec((_MMB, _D), lambda i: (i, 0)),
            pl.BlockSpec((_D, _D), lambda i: (0, 0)),
            pl.BlockSpec((1, _D), lambda i: (0, 0)),
        ],
        out_specs=pl.BlockSpec((_MMB, _D), lambda i: (i, 0)),
        out_shape=jax.ShapeDtypeStruct((_N, _D), jnp.float32),
    )(emb, w, b2)


def _rr(work_pred, nchunks, body):
    tid = lax.axis_index("s")
    nloop = (nchunks + _NS - 1) // _NS
    for jj in range(nloop):
        k = tid + jj * _NS
        if (jj + 1) * _NS <= nchunks:
            pl.when(work_pred)(lambda k=k: body(k))
        else:
            pl.when(jnp.logical_and(work_pred, k < nchunks))(
                lambda k=k: body(k))


_PMAXROWS = 8192


@functools.partial(
    pl.kernel,
    out_type=jax.ShapeDtypeStruct((_N, _D), jnp.float32),
    mesh=_mesh(),
    scratch_types=[
        pltpu.VMEM((_GCH,), jnp.int32),
        pltpu.VMEM((_GCH, _D), jnp.float32),
        pltpu.VMEM((_PCH, _D), jnp.float32),
        pltpu.VMEM_SHARED((_PMAXROWS + 8, _D), jnp.float32),
    ],
)
def _tree(h_hbm, pp_hbm, hout_hbm, idx1d, childbuf, pbuf, partial):
    cid = lax.axis_index("c")
    tid = lax.axis_index("s")
    on0 = cid == 0
    on1 = cid == 1

    def stage_copy(src, soff, dst, doff, rows):
        pltpu.sync_copy(src.at[pl.ds(soff, rows)], pbuf.at[pl.ds(0, rows)])
        pltpu.sync_copy(pbuf.at[pl.ds(0, rows)], dst.at[pl.ds(doff, rows)])

    nlev = len(_LEVEL_SIZES)
    for l in range(nlev - 1, 0, -1):
        s, n = _OFFS[l], _LEVEL_SIZES[l]
        ps, pn = _OFFS[l - 1], _LEVEL_SIZES[l - 1]
        src = h_hbm if l == nlev - 1 else hout_hbm
        if pn > _PMAXROWS:
            passes = [(0, pn // 2), (pn // 2, pn // 2)]
        else:
            passes = [(0, pn)]

        for hp, (lo, np_) in enumerate(passes):
            clamp = len(passes) > 1

            nfull, tail = np_ // _PCH, np_ % _PCH
            _rr(on0, nfull,
                lambda k, b=ps + lo: stage_copy(h_hbm, b + k * _PCH, partial,
                                                k * _PCH, _PCH))
            if tail:
                pl.when(jnp.logical_and(on0, tid == (nfull % _NS)))(
                    lambda b=ps + lo, o=nfull * _PCH, t=tail:
                    stage_copy(h_hbm, b + o, partial, o, t))
            if l == nlev - 1 and hp == 0:
                lb, ln = _OFFS[nlev - 1], _LEVEL_SIZES[nlev - 1]
                _rr(on1, ln // _PCH,
                    lambda k, lb=lb: stage_copy(h_hbm, lb + k * _PCH,
                                                hout_hbm, lb + k * _PCH,
                                                _PCH))
            plsc.subcore_barrier()

            def p2_full(k, s=s, src=src, lo=lo, np_=np_, clamp=clamp):
                coff = s + k * _GCH
                pltpu.sync_copy(pp_hbm.at[pl.ds(coff, _GCH)], idx1d)
                if clamp:
                    for q in range(_GCH // 16):
                        v = idx1d[pl.ds(q * 16, 16)]
                        loc = v - lo
                        ok = jnp.logical_and(v >= lo, v < lo + np_)
                        idx1d[pl.ds(q * 16, 16)] = jnp.where(
                            ok, loc, jnp.int32(np_))
                pltpu.sync_copy(src.at[pl.ds(coff, _GCH)], childbuf)
                pltpu.sync_copy(childbuf, partial.at[idx1d], add=True)

            cfull, ctail = n // _GCH, n % _GCH
            _rr(on0, cfull, p2_full)
            if ctail:
                def p2_tail(s=s, src=src, k=cfull, cnt=ctail):
                    def scoped(idxs):
                        coff = s + k * _GCH
                        pltpu.sync_copy(pp_hbm.at[pl.ds(coff, cnt)], idxs)
                        pltpu.sync_copy(src.at[pl.ds(coff, cnt)],
                                        childbuf.at[pl.ds(0, cnt)])
                        pltpu.sync_copy(childbuf.at[pl.ds(0, cnt)],
                                        partial.at[idxs], add=True)
                    pl.run_scoped(scoped, pltpu.VMEM((cnt,), jnp.int32))
                pl.when(jnp.logical_and(on0, tid == (cfull % _NS)))(p2_tail)
            plsc.subcore_barrier()

            _rr(on0, nfull,
                lambda k, b=ps + lo: stage_copy(partial, k * _PCH, hout_hbm,
                                                b + k * _PCH, _PCH))
            if tail:
                pl.when(jnp.logical_and(on0, tid == (nfull % _NS)))(
                    lambda b=ps + lo, o=nfull * _PCH, t=tail:
                    stage_copy(partial, o, hout_hbm, b + o, t))
            plsc.subcore_barrier()


def _segmax(bi2, hout):
    def body(ids_ref, x_ref, o_ref):
        pid = pl.program_id(0)

        @pl.when(pid == 0)
        def _():
            o_ref[...] = jnp.zeros((_BS, _D), jnp.float32)

        x = x_ref[...]
        ids = ids_ref[...]
        parts = []
        for s2 in range(_BS):
            parts.append(jnp.max(jnp.where(ids == s2, x, -1e30), axis=0,
                                 keepdims=True))
        o_ref[...] = jnp.maximum(o_ref[...], jnp.concatenate(parts, axis=0))

    return pl.pallas_call(
        body,
        grid=(_N // _MMB,),
        in_specs=[
            pl.BlockSpec((_MMB, 1), lambda i: (i, 0)),
            pl.BlockSpec((_MMB, _D), lambda i: (i, 0)),
        ],
        out_specs=pl.BlockSpec((_BS, _D), lambda i: (0, 0)),
        out_shape=jax.ShapeDtypeStruct((_BS, _D), jnp.float32),
    )(bi2, hout)


def kernel(tokens, level_offsets, parent_pos, batch_index, bs, table, W, b):
    del level_offsets, bs
    tok2 = tokens.astype(jnp.int32).reshape(_N // _GCH, _GCH)
    pp32 = parent_pos.astype(jnp.int32)
    bi2 = batch_index.astype(jnp.int32).reshape(_N, 1)
    emb = _gather(table, tok2)
    h = _matmul(emb, W, b.reshape(1, _D))
    hout = _tree(h, pp32)
    return _segmax(bi2, hout)


# speedup vs baseline: 2.9466x; 1.2119x over previous
import functools

import jax
import jax.numpy as jnp
from jax import lax
from jax.experimental import pallas as pl
from jax.experimental.pallas import tpu as pltpu
from jax.experimental.pallas import tpu_sc as plsc

_LEVEL_SIZES = (16, 48, 192, 768, 3072, 8192, 10240, 10240)
_N = sum(_LEVEL_SIZES)
_D = 128
_BS = 16
_NC = 2
_NS = 16
_NW = _NC * _NS

_OFFS = [0]
for _s in _LEVEL_SIZES:
    _OFFS.append(_OFFS[-1] + _s)

_GCH = 128
_ROWS_PER_TILE = _N // _NW
_GN = _ROWS_PER_TILE // _GCH
_PCH = 256
_MMB = 512


def _mesh():
    return plsc.VectorSubcoreMesh(core_axis_name="c", subcore_axis_name="s")


@functools.partial(
    pl.kernel,
    out_type=jax.ShapeDtypeStruct((_N, _D), jnp.float32),
    mesh=_mesh(),
    scratch_types=[
        pltpu.VMEM((_GN, _GCH), jnp.int32),
        pltpu.VMEM((2, _GCH, _D), jnp.float32),
        pltpu.SemaphoreType.DMA,
        pltpu.SemaphoreType.DMA,
    ],
)
def _gather(table_hbm, tok_hbm, out_hbm, idx_v, bufs, sem0, sem1):
    wid = lax.axis_index("s") * _NC + lax.axis_index("c")
    pltpu.sync_copy(tok_hbm.at[pl.ds(wid * _GN, _GN)], idx_v)
    sems = (sem0, sem1)
    cps = [None, None]
    cps[0] = pltpu.async_copy(table_hbm.at[idx_v.at[0]], bufs.at[0], sem0)
    for c in range(_GN):
        cur = c % 2
        if c + 1 < _GN:
            cps[1 - cur] = pltpu.async_copy(
                table_hbm.at[idx_v.at[c + 1]], bufs.at[1 - cur], sems[1 - cur])
        cps[cur].wait()
        pltpu.sync_copy(
            bufs.at[cur],
            out_hbm.at[pl.ds(wid * _ROWS_PER_TILE + c * _GCH, _GCH)])


def _matmul(emb, w, b2):
    def body(x_ref, w_ref, b_ref, o_ref):
        o_ref[...] = lax.dot_general(
            x_ref[...], w_ref[...], (((1,), (1,)), ((), ())),
            preferred_element_type=jnp.float32) + b_ref[...]

    return pl.pallas_call(
        body,
        grid=(_N // _MMB,),
        in_specs=[
            pl.BlockSpec((_MMB, _D), lambda i: (i, 0)),
            pl.BlockSpec((_D, _D), lambda i: (0, 0)),
            pl.BlockSpec((1, _D), lambda i: (0, 0)),
        ],
        out_specs=pl.BlockSpec((_MMB, _D), lambda i: (i, 0)),
        out_shape=jax.ShapeDtypeStruct((_N, _D), jnp.float32),
    )(emb, w, b2)


_PMAXROWS = 5120
_NINT = 22528


@functools.partial(
    pl.kernel,
    out_type=jax.ShapeDtypeStruct((_NINT, _D), jnp.float32),
    mesh=_mesh(),
    scratch_types=[
        pltpu.VMEM((5, _GCH), jnp.int32),
        pltpu.VMEM((640, _D), jnp.float32),
        pltpu.VMEM_SHARED((_PMAXROWS + 8, _D), jnp.float32),
        pltpu.SemaphoreType.DMA,
    ],
)
def _tree(h_hbm, pp_hbm, hout_hbm, idx2d, buf, partial, sem):
    cid = lax.axis_index("c")
    tid = lax.axis_index("s")
    on0 = cid == 0

    nlev = len(_LEVEL_SIZES)
    for l in range(nlev - 1, 0, -1):
        s, n = _OFFS[l], _LEVEL_SIZES[l]
        ps, pn = _OFFS[l - 1], _LEVEL_SIZES[l - 1]
        src = h_hbm if l == nlev - 1 else hout_hbm
        if pn > _PMAXROWS:
            passes = [(0, pn // 2), (pn // 2, pn // 2)]
        else:
            passes = [(0, pn)]
        A = min(_NS, n // _GCH) if n >= _GCH else 1
        span = n // A
        full, tail = span // _GCH, span % _GCH

        for lo, np_ in passes:
            clamp = len(passes) > 1
            ap = max(a for a in range(1, _NS + 1)
                     if np_ % a == 0 and (np_ // a) % 8 == 0)
            pspan = np_ // ap

            def prow(ref, base):
                off = pl.multiple_of(base + tid * pspan, 8)
                return ref.at[pl.ds(off, pspan)]

            def srow():
                off = pl.multiple_of(tid * pspan, 8)
                return partial.at[pl.ds(off, pspan)]

            bslice = buf.at[pl.ds(0, pspan)]

            pon = jnp.logical_and(on0, tid < ap) if ap < _NS else on0

            def p1(b=ps + lo, prow=prow, srow=srow, bslice=bslice):
                pltpu.sync_copy(prow(h_hbm, b), bslice)
                pltpu.sync_copy(bslice, srow())

            pl.when(pon)(p1)
            plsc.subcore_barrier()

            def p2(s=s, src=src, lo=lo, np_=np_, clamp=clamp, span=span,
                   full=full, tail=tail):
                cbase = pl.multiple_of(s + tid * span, 8)
                ppcps = [pltpu.async_copy(
                    pp_hbm.at[pl.ds(cbase + j * _GCH, _GCH)], idx2d.at[j],
                    sem) for j in range(full)]
                pltpu.sync_copy(src.at[pl.ds(cbase, span)],
                                buf.at[pl.ds(0, span)])
                for cp in ppcps:
                    cp.wait()
                if clamp:
                    for j in range(full):
                        for q in range(_GCH // 16):
                            v = idx2d[j, pl.ds(q * 16, 16)]
                            ok = jnp.logical_and(v >= lo, v < lo + np_)
                            idx2d[j, pl.ds(q * 16, 16)] = jnp.where(
                                ok, v - lo, jnp.int32(np_))
                for j in range(full):
                    pltpu.sync_copy(buf.at[pl.ds(j * _GCH, _GCH)],
                                    partial.at[idx2d.at[j]], add=True)
                if tail:
                    def scoped(idxs):
                        pltpu.sync_copy(
                            pp_hbm.at[pl.ds(cbase + full * _GCH, tail)], idxs)
                        pltpu.sync_copy(buf.at[pl.ds(full * _GCH, tail)],
                                        partial.at[idxs], add=True)
                    pl.run_scoped(scoped, pltpu.VMEM((tail,), jnp.int32))

            pl.when(jnp.logical_and(on0, tid < A) if A < _NS else on0)(p2)
            plsc.subcore_barrier()

            def p3(b=ps + lo, prow=prow, srow=srow, bslice=bslice):
                pltpu.sync_copy(srow(), bslice)
                pltpu.sync_copy(bslice, prow(hout_hbm, b))

            pl.when(pon)(p3)
            plsc.subcore_barrier()


def _segmax(bi2, x, init, base_blk, nrows):
    def body(ids_ref, x_ref, init_ref, o_ref):
        pid = pl.program_id(0)

        @pl.when(pid == 0)
        def _():
            o_ref[...] = init_ref[...]

        x_ = x_ref[...]
        ids = ids_ref[...]
        parts = []
        for s2 in range(_BS):
            parts.append(jnp.max(jnp.where(ids == s2, x_, -1e30), axis=0,
                                 keepdims=True))
        o_ref[...] = jnp.maximum(o_ref[...], jnp.concatenate(parts, axis=0))

    return pl.pallas_call(
        body,
        grid=(nrows // _MMB,),
        in_specs=[
            pl.BlockSpec((_MMB, 1), lambda i, b=base_blk: (b + i, 0)),
            pl.BlockSpec((_MMB, _D), lambda i, b=base_blk: (b + i, 0)),
            pl.BlockSpec((_BS, _D), lambda i: (0, 0)),
        ],
        out_specs=pl.BlockSpec((_BS, _D), lambda i: (0, 0)),
        out_shape=jax.ShapeDtypeStruct((_BS, _D), jnp.float32),
    )(bi2, x, init)


def kernel(tokens, level_offsets, parent_pos, batch_index, bs, table, W, b):
    del level_offsets, bs
    tok2 = tokens.astype(jnp.int32).reshape(_N // _GCH, _GCH)
    pp32 = parent_pos.astype(jnp.int32)
    bi2 = batch_index.astype(jnp.int32).reshape(_N, 1)
    emb = _gather(table, tok2)
    h = _matmul(emb, W, b.reshape(1, _D))
    hout = _tree(h, pp32)
    part = _segmax(bi2, h, jnp.zeros((_BS, _D), jnp.float32),
                   _NINT // _MMB, _N - _NINT)
    return _segmax(bi2, hout, part, 0, _NINT)
